# baseline (speedup 1.0000x reference)
import functools

import jax
import jax.numpy as jnp
from jax import lax
from jax.experimental import pallas as pl
from jax.experimental.pallas import tpu as pltpu
from jax.experimental.pallas import tpu_sc as plsc

NC = 2
NS = 16
NW = NC * NS
CHUNK = 80
ROW_BLK = 1024


def _count_body(edges_hbm, out_hbm, src_v, dst_v, ones_v, scnt, dcnt, sems,
                semd, *, nch, n_pad):
    cid = lax.axis_index("c")
    sid = lax.axis_index("s")
    wid = cid * NS + sid
    rps = n_pad // NS
    zblk = 2 * CHUNK

    for k in range(zblk // 16):
        ones_v[pl.ds(k * 16, 16)] = jnp.zeros((16,), jnp.float32)

    def zcopy(r, carry):
        pltpu.sync_copy(ones_v, scnt.at[pl.ds(sid * rps + r * zblk, zblk)])
        pltpu.sync_copy(ones_v, dcnt.at[pl.ds(sid * rps + r * zblk, zblk)])
        return carry

    lax.fori_loop(0, rps // zblk, zcopy, 0)
    for k in range(CHUNK // 16):
        ones_v[pl.ds(k * 16, 16)] = jnp.full((16,), 1.0, jnp.float32)
    pltpu.sync_copy(edges_hbm.at[0, wid], src_v)
    pltpu.sync_copy(edges_hbm.at[1, wid], dst_v)
    plsc.subcore_barrier()
    ones = ones_v.at[pl.ds(0, CHUNK)]

    def step(j, carry):
        pltpu.async_copy(ones, scnt.at[src_v.at[j]], sems, add=True)
        pltpu.async_copy(ones, dcnt.at[dst_v.at[j]], semd, add=True)

        @pl.when(j > 0)
        def _():
            pltpu.make_async_copy(ones, scnt.at[src_v.at[j - 1]],
                                  sems).wait()
            pltpu.make_async_copy(ones, dcnt.at[dst_v.at[j - 1]],
                                  semd).wait()

        return carry

    lax.fori_loop(0, nch, step, 0)
    pltpu.make_async_copy(ones, scnt.at[src_v.at[nch - 1]], sems).wait()
    pltpu.make_async_copy(ones, dcnt.at[dst_v.at[nch - 1]], semd).wait()
    plsc.subcore_barrier()
    pltpu.sync_copy(scnt.at[pl.ds(sid * rps, rps)],
                    out_hbm.at[cid, 0, pl.ds(sid * rps, rps)])
    pltpu.sync_copy(dcnt.at[pl.ds(sid * rps, rps)],
                    out_hbm.at[cid, 1, pl.ds(sid * rps, rps)])


def _agg_body(featsrc_hbm, edges_hbm, out_hbm,
              src_v, dst_v, b0, b1, acc, g0, g1, s0, s1,
              *, nch, nwin, n_pad, d):
    cid = lax.axis_index("c")
    sid = lax.axis_index("s")
    wid = cid * NS + sid
    rps = n_pad // NS
    wch = nch // nwin
    npair = wch // 2

    def zrow(i, carry):
        for k in range(d // 16):
            b0[i, pl.ds(k * 16, 16)] = jnp.zeros((16,), jnp.float32)
        return carry

    lax.fori_loop(0, CHUNK, zrow, 0)
    for r in range(rps // CHUNK):
        pltpu.sync_copy(b0, acc.at[pl.ds(sid * rps + r * CHUNK, CHUNK)])
    plsc.subcore_barrier()

    def gather(j, buf, sem):
        return pltpu.async_copy(featsrc_hbm.at[src_v.at[j]], buf, sem)

    def scatter(j, buf, sem):
        return pltpu.async_copy(buf, acc.at[dst_v.at[j]], sem, add=True)

    def window(w, carry):
        pltpu.sync_copy(edges_hbm.at[0, wid, w], src_v)
        pltpu.sync_copy(edges_hbm.at[1, wid, w], dst_v)
        gather(0, b0, g0)

        def pair(t, carry2):
            j0 = 2 * t
            j1 = 2 * t + 1
            pltpu.make_async_copy(featsrc_hbm.at[src_v.at[j0]], b0,
                                  g0).wait()
            scatter(j0, b0, s0)

            @pl.when(t > 0)
            def _():
                pltpu.make_async_copy(b1, acc.at[dst_v.at[j1 - 2]],
                                      s1).wait()

            gather(j1, b1, g1)
            pltpu.make_async_copy(featsrc_hbm.at[src_v.at[j1]], b1,
                                  g1).wait()
            scatter(j1, b1, s1)
            pltpu.make_async_copy(b0, acc.at[dst_v.at[j0]], s0).wait()

            @pl.when(t < npair - 1)
            def _():
                gather(j0 + 2, b0, g0)

            return carry2

        lax.fori_loop(0, npair, pair, 0)
        jt = wch - 1
        gather(jt, b0, g0)
        pltpu.make_async_copy(featsrc_hbm.at[src_v.at[jt]], b0, g0).wait()
        scatter(jt, b0, s0)
        pltpu.make_async_copy(b1, acc.at[dst_v.at[jt - 1]], s1).wait()
        pltpu.make_async_copy(b0, acc.at[dst_v.at[jt]], s0).wait()
        return carry

    lax.fori_loop(0, nwin, window, 0)
    plsc.subcore_barrier()
    pltpu.sync_copy(acc.at[pl.ds(sid * rps, rps)],
                    out_hbm.at[cid, pl.ds(sid * rps, rps)])


def _scale_body(cnt_ref, feat_ref, wt_ref, out_ref):
    src_cnt = cnt_ref[0, 0, :] + cnt_ref[1, 0, :]
    ns = lax.rsqrt(jnp.maximum(src_cnt, 1.0))
    y = jnp.dot(feat_ref[...], wt_ref[...],
                preferred_element_type=jnp.float32)
    out_ref[...] = y * ns[:, None]


def _final_body(agg_ref, cnt_ref, feat_ref, b_ref, out_ref):
    a = agg_ref[0] + agg_ref[1]
    dst_cnt = cnt_ref[0, 1, :] + cnt_ref[1, 1, :]
    nd = lax.rsqrt(jnp.maximum(dst_cnt, 1.0))[:, None]
    out_ref[...] = (a + b_ref[...]) * nd + feat_ref[...]


def kernel(feat, edge_index, W, b):
    n, d = feat.shape
    e = edge_index.shape[1]

    n_pad = -(-(n + 1) // ROW_BLK) * ROW_BLK
    assert e % (NW * CHUNK) == 0, "edge count must tile over 32x80"
    nch = e // (NW * CHUNK)

    nwin = 5
    assert nch % nwin == 0 and (nch // nwin) % 2 == 1
    edges = edge_index.astype(jnp.int32).reshape(2, NW, nch, CHUNK)
    edges_w = edges.reshape(2, NW, nwin, nch // nwin, CHUNK)

    mesh = plsc.VectorSubcoreMesh(core_axis_name="c", subcore_axis_name="s")

    count_k = pl.kernel(
        functools.partial(_count_body, nch=nch, n_pad=n_pad),
        out_type=jax.ShapeDtypeStruct((NC, 2, n_pad), jnp.float32),
        mesh=mesh,
        scratch_types=[
            pltpu.VMEM((nch, CHUNK), jnp.int32),
            pltpu.VMEM((nch, CHUNK), jnp.int32),
            pltpu.VMEM((2 * CHUNK,), jnp.float32),
            pltpu.VMEM_SHARED((n_pad,), jnp.float32),
            pltpu.VMEM_SHARED((n_pad,), jnp.float32),
            pltpu.SemaphoreType.DMA,
            pltpu.SemaphoreType.DMA,
        ],
    )
    cnt = count_k(edges)

    grid = n_pad // ROW_BLK
    feat_src = pl.pallas_call(
        _scale_body,
        grid=(grid,),
        in_specs=[
            pl.BlockSpec((NC, 2, ROW_BLK), lambda i: (0, 0, i)),
            pl.BlockSpec((ROW_BLK, d), lambda i: (i, 0)),
            pl.BlockSpec((d, d), lambda i: (0, 0)),
        ],
        out_specs=pl.BlockSpec((ROW_BLK, d), lambda i: (i, 0)),
        out_shape=jax.ShapeDtypeStruct((n_pad, d), jnp.float32),
    )(cnt, feat, W.T)

    agg_k = pl.kernel(
        functools.partial(_agg_body, nch=nch, nwin=nwin, n_pad=n_pad, d=d),
        out_type=jax.ShapeDtypeStruct((NC, n_pad, d), jnp.float32),
        mesh=mesh,
        scratch_types=[
            pltpu.VMEM((nch // nwin, CHUNK), jnp.int32),
            pltpu.VMEM((nch // nwin, CHUNK), jnp.int32),
            pltpu.VMEM((CHUNK, d), jnp.float32),
            pltpu.VMEM((CHUNK, d), jnp.float32),
            pltpu.VMEM_SHARED((n_pad, d), jnp.float32),
            pltpu.SemaphoreType.DMA,
            pltpu.SemaphoreType.DMA,
            pltpu.SemaphoreType.DMA,
            pltpu.SemaphoreType.DMA,
        ],
    )
    agg = agg_k(feat_src, edges_w)

    return pl.pallas_call(
        _final_body,
        grid=(grid,),
        in_specs=[
            pl.BlockSpec((NC, ROW_BLK, d), lambda i: (0, i, 0)),
            pl.BlockSpec((NC, 2, ROW_BLK), lambda i: (0, 0, i)),
            pl.BlockSpec((ROW_BLK, d), lambda i: (i, 0)),
            pl.BlockSpec((1, d), lambda i: (0, 0)),
        ],
        out_specs=pl.BlockSpec((ROW_BLK, d), lambda i: (i, 0)),
        out_shape=jax.ShapeDtypeStruct((n, d), jnp.float32),
    )(agg, cnt, feat, b.reshape(1, d))


# speedup vs baseline: 1.2099x; 1.2099x over previous
import functools

import jax
import jax.numpy as jnp
from jax import lax
from jax.experimental import pallas as pl
from jax.experimental.pallas import tpu as pltpu
from jax.experimental.pallas import tpu_sc as plsc

NC = 2
NS = 16
NW = NC * NS
CHUNK = 80
NBUF = 4
ROW_BLK = 1024


def _count_body(edges_hbm, out_hbm, src_v, dst_v, ones_v, scnt, dcnt, sems,
                semd, *, nwin, wch, n_pad):
    cid = lax.axis_index("c")
    sid = lax.axis_index("s")
    wid = cid * NS + sid
    rps = n_pad // NS
    zblk = 2 * CHUNK

    for k in range(zblk // 16):
        ones_v[pl.ds(k * 16, 16)] = jnp.zeros((16,), jnp.float32)

    def zcopy(r, carry):
        pltpu.sync_copy(ones_v, scnt.at[pl.ds(sid * rps + r * zblk, zblk)])
        pltpu.sync_copy(ones_v, dcnt.at[pl.ds(sid * rps + r * zblk, zblk)])
        return carry

    lax.fori_loop(0, rps // zblk, zcopy, 0)
    for k in range(CHUNK // 16):
        ones_v[pl.ds(k * 16, 16)] = jnp.full((16,), 1.0, jnp.float32)
    pltpu.sync_copy(edges_hbm.at[0, wid], src_v)
    pltpu.sync_copy(edges_hbm.at[1, wid], dst_v)
    plsc.subcore_barrier()
    ones = ones_v.at[pl.ds(0, CHUNK)]

    def step(t, carry):
        w = t // wch
        j = t % wch
        pltpu.async_copy(ones, scnt.at[src_v.at[w, j]], sems, add=True)
        pltpu.async_copy(ones, dcnt.at[dst_v.at[w, j]], semd, add=True)

        @pl.when(t > 0)
        def _():
            wp = (t - 1) // wch
            jp = (t - 1) % wch
            pltpu.make_async_copy(ones, scnt.at[src_v.at[wp, jp]],
                                  sems).wait()
            pltpu.make_async_copy(ones, dcnt.at[dst_v.at[wp, jp]],
                                  semd).wait()

        return carry

    nch = nwin * wch
    lax.fori_loop(0, nch, step, 0)
    pltpu.make_async_copy(ones, scnt.at[src_v.at[nwin - 1, wch - 1]],
                          sems).wait()
    pltpu.make_async_copy(ones, dcnt.at[dst_v.at[nwin - 1, wch - 1]],
                          semd).wait()
    plsc.subcore_barrier()
    pltpu.sync_copy(scnt.at[pl.ds(sid * rps, rps)],
                    out_hbm.at[cid, 0, pl.ds(sid * rps, rps)])
    pltpu.sync_copy(dcnt.at[pl.ds(sid * rps, rps)],
                    out_hbm.at[cid, 1, pl.ds(sid * rps, rps)])


def _agg_body(featsrc_hbm, edges_hbm, out_hbm,
              src_v, dst_v, b0, b1, b2, b3, acc,
              g0, g1, g2, g3, s0, s1, s2, s3,
              *, nwin, wch, n_acc, d):
    cid = lax.axis_index("c")
    sid = lax.axis_index("s")
    wid = cid * NS + sid
    rps = n_acc // NS
    bufs = (b0, b1, b2, b3)
    gsem = (g0, g1, g2, g3)
    ssem = (s0, s1, s2, s3)
    nround = wch // NBUF

    def zrow(i, carry):
        for k in range(d // 16):
            b0[i, pl.ds(k * 16, 16)] = jnp.zeros((16,), jnp.float32)
        return carry

    lax.fori_loop(0, CHUNK, zrow, 0)
    for r in range(rps // CHUNK):
        pltpu.sync_copy(b0, acc.at[pl.ds(sid * rps + r * CHUNK, CHUNK)])
    rem = rps % CHUNK
    if rem:
        pltpu.sync_copy(
            b0.at[pl.ds(0, rem)],
            acc.at[pl.ds(sid * rps + (rps // CHUNK) * CHUNK, rem)])
    plsc.subcore_barrier()

    def gather(j, buf, sem):
        return pltpu.async_copy(featsrc_hbm.at[src_v.at[j]], buf, sem)

    def gwait(j, buf, sem):
        pltpu.make_async_copy(featsrc_hbm.at[src_v.at[j]], buf, sem).wait()

    def scatter(j, buf, sem):
        return pltpu.async_copy(buf, acc.at[dst_v.at[j]], sem, add=True)

    def swait(j, buf, sem):
        pltpu.make_async_copy(buf, acc.at[dst_v.at[j]], sem).wait()

    def window(w, carry):
        pltpu.sync_copy(edges_hbm.at[0, wid, w], src_v)
        pltpu.sync_copy(edges_hbm.at[1, wid, w], dst_v)
        for k in range(NBUF):
            gather(k, bufs[k], gsem[k])

        def rnd(r, carry2):
            for k in range(NBUF):
                j = r * NBUF + k
                gwait(j, bufs[k], gsem[k])
                scatter(j, bufs[k], ssem[k])
            for k in range(NBUF):
                j = r * NBUF + k
                swait(j, bufs[k], ssem[k])

                @pl.when(r < nround - 1)
                def _():
                    gather(j + NBUF, bufs[k], gsem[k])

            return carry2

        lax.fori_loop(0, nround, rnd, 0)
        jt = wch - 1
        gather(jt, b0, g0)
        gwait(jt, b0, g0)
        scatter(jt, b0, s0)
        swait(jt, b0, s0)
        return carry

    lax.fori_loop(0, nwin, window, 0)
    plsc.subcore_barrier()
    pltpu.sync_copy(acc.at[pl.ds(sid * rps, rps)],
                    out_hbm.at[cid, pl.ds(sid * rps, rps)])


def _scale_body(cnt_ref, feat_ref, wt_ref, out_ref):
    src_cnt = cnt_ref[0, 0, :] + cnt_ref[1, 0, :]
    ns = lax.rsqrt(jnp.maximum(src_cnt, 1.0))
    y = jnp.dot(feat_ref[...], wt_ref[...],
                preferred_element_type=jnp.float32)
    out_ref[...] = y * ns[:, None]


def _final_body(agg_ref, cnt_ref, feat_ref, b_ref, out_ref):
    a = agg_ref[0] + agg_ref[1]
    dst_cnt = cnt_ref[0, 1, :] + cnt_ref[1, 1, :]
    nd = lax.rsqrt(jnp.maximum(dst_cnt, 1.0))[:, None]
    out_ref[...] = (a + b_ref[...]) * nd + feat_ref[...]


def kernel(feat, edge_index, W, b):
    n, d = feat.shape
    e = edge_index.shape[1]

    n_pad = -(-(n + 1) // ROW_BLK) * ROW_BLK
    assert e % (NW * CHUNK) == 0, "edge count must tile over 32x80"
    nch = e // (NW * CHUNK)
    n_acc = -(-(n + 1) // 128) * 128

    nwin = 5
    assert nch % nwin == 0 and (nch // nwin) % NBUF == 1
    wch = nch // nwin
    edges = edge_index.astype(jnp.int32).reshape(2, NW, nwin, wch, CHUNK)

    mesh = plsc.VectorSubcoreMesh(core_axis_name="c", subcore_axis_name="s")

    count_k = pl.kernel(
        functools.partial(_count_body, nwin=nwin, wch=wch, n_pad=n_pad),
        out_type=jax.ShapeDtypeStruct((NC, 2, n_pad), jnp.float32),
        mesh=mesh,
        scratch_types=[
            pltpu.VMEM((nwin, wch, CHUNK), jnp.int32),
            pltpu.VMEM((nwin, wch, CHUNK), jnp.int32),
            pltpu.VMEM((2 * CHUNK,), jnp.float32),
            pltpu.VMEM_SHARED((n_pad,), jnp.float32),
            pltpu.VMEM_SHARED((n_pad,), jnp.float32),
            pltpu.SemaphoreType.DMA,
            pltpu.SemaphoreType.DMA,
        ],
    )
    cnt = count_k(edges)

    grid = n_pad // ROW_BLK
    feat_src = pl.pallas_call(
        _scale_body,
        grid=(grid,),
        in_specs=[
            pl.BlockSpec((NC, 2, ROW_BLK), lambda i: (0, 0, i)),
            pl.BlockSpec((ROW_BLK, d), lambda i: (i, 0)),
            pl.BlockSpec((d, d), lambda i: (0, 0)),
        ],
        out_specs=pl.BlockSpec((ROW_BLK, d), lambda i: (i, 0)),
        out_shape=jax.ShapeDtypeStruct((n_pad, d), jnp.float32),
    )(cnt, feat, W.T)

    agg_k = pl.kernel(
        functools.partial(_agg_body, nwin=nwin, wch=wch, n_acc=n_acc, d=d),
        out_type=jax.ShapeDtypeStruct((NC, n_acc, d), jnp.float32),
        mesh=mesh,
        scratch_types=[
            pltpu.VMEM((wch, CHUNK), jnp.int32),
            pltpu.VMEM((wch, CHUNK), jnp.int32),
            pltpu.VMEM((CHUNK, d), jnp.float32),
            pltpu.VMEM((CHUNK, d), jnp.float32),
            pltpu.VMEM((CHUNK, d), jnp.float32),
            pltpu.VMEM((CHUNK, d), jnp.float32),
            pltpu.VMEM_SHARED((n_acc, d), jnp.float32),
            pltpu.SemaphoreType.DMA,
            pltpu.SemaphoreType.DMA,
            pltpu.SemaphoreType.DMA,
            pltpu.SemaphoreType.DMA,
            pltpu.SemaphoreType.DMA,
            pltpu.SemaphoreType.DMA,
            pltpu.SemaphoreType.DMA,
            pltpu.SemaphoreType.DMA,
        ],
    )
    agg = agg_k(feat_src, edges)

    return pl.pallas_call(
        _final_body,
        grid=(grid,),
        in_specs=[
            pl.BlockSpec((NC, ROW_BLK, d), lambda i: (0, i, 0)),
            pl.BlockSpec((NC, 2, ROW_BLK), lambda i: (0, 0, i)),
            pl.BlockSpec((ROW_BLK, d), lambda i: (i, 0)),
            pl.BlockSpec((1, d), lambda i: (0, 0)),
        ],
        out_specs=pl.BlockSpec((ROW_BLK, d), lambda i: (i, 0)),
        out_shape=jax.ShapeDtypeStruct((n, d), jnp.float32),
    )(agg, cnt, feat, b.reshape(1, d))


# speedup vs baseline: 1.2360x; 1.0216x over previous
import functools

import jax
import jax.numpy as jnp
from jax import lax
from jax.experimental import pallas as pl
from jax.experimental.pallas import tpu as pltpu
from jax.experimental.pallas import tpu_sc as plsc

NC = 2
NS = 16
NW = NC * NS
CHUNK = 80
NBUF = 4
ROW_BLK = 1024


def _count_body(edges_hbm, out_hbm, src_v, dst_v, ones_v, scnt, dcnt, sems,
                semd, *, nwin, wch, n_pad):
    cid = lax.axis_index("c")
    sid = lax.axis_index("s")
    wid = cid * NS + sid
    rps = n_pad // NS
    zblk = 2 * CHUNK

    for k in range(zblk // 16):
        ones_v[pl.ds(k * 16, 16)] = jnp.zeros((16,), jnp.float32)

    def zcopy(r, carry):
        pltpu.sync_copy(ones_v, scnt.at[pl.ds(sid * rps + r * zblk, zblk)])
        pltpu.sync_copy(ones_v, dcnt.at[pl.ds(sid * rps + r * zblk, zblk)])
        return carry

    lax.fori_loop(0, rps // zblk, zcopy, 0)
    for k in range(CHUNK // 16):
        ones_v[pl.ds(k * 16, 16)] = jnp.full((16,), 1.0, jnp.float32)
    pltpu.sync_copy(edges_hbm.at[0, wid], src_v)
    pltpu.sync_copy(edges_hbm.at[1, wid], dst_v)
    plsc.subcore_barrier()
    ones = ones_v.at[pl.ds(0, CHUNK)]

    def step(t, carry):
        w = t // wch
        j = t % wch
        pltpu.async_copy(ones, scnt.at[src_v.at[w, j]], sems, add=True)
        pltpu.async_copy(ones, dcnt.at[dst_v.at[w, j]], semd, add=True)

        @pl.when(t > 0)
        def _():
            wp = (t - 1) // wch
            jp = (t - 1) % wch
            pltpu.make_async_copy(ones, scnt.at[src_v.at[wp, jp]],
                                  sems).wait()
            pltpu.make_async_copy(ones, dcnt.at[dst_v.at[wp, jp]],
                                  semd).wait()

        return carry

    nch = nwin * wch
    lax.fori_loop(0, nch, step, 0)
    pltpu.make_async_copy(ones, scnt.at[src_v.at[nwin - 1, wch - 1]],
                          sems).wait()
    pltpu.make_async_copy(ones, dcnt.at[dst_v.at[nwin - 1, wch - 1]],
                          semd).wait()
    plsc.subcore_barrier()
    pltpu.sync_copy(scnt.at[pl.ds(sid * rps, rps)],
                    out_hbm.at[cid, 0, pl.ds(sid * rps, rps)])
    pltpu.sync_copy(dcnt.at[pl.ds(sid * rps, rps)],
                    out_hbm.at[cid, 1, pl.ds(sid * rps, rps)])


def _agg_body(featsrc_hbm, edges_hbm, out_hbm,
              src_v, dst_v, b0, b1, b2, b3, acc,
              g0, g1, g2, g3, s0, s1, s2, s3,
              *, nwin, wch, n_acc, d):
    cid = lax.axis_index("c")
    sid = lax.axis_index("s")
    wid = cid * NS + sid
    rps = n_acc // NS
    bufs = (b0, b1, b2, b3)
    gsem = (g0, g1, g2, g3)
    ssem = (s0, s1, s2, s3)
    nround = wch // NBUF

    def zrow(i, carry):
        for k in range(d // 16):
            b0[i, pl.ds(k * 16, 16)] = jnp.zeros((16,), jnp.float32)
        return carry

    lax.fori_loop(0, CHUNK, zrow, 0)
    for r in range(rps // CHUNK):
        pltpu.sync_copy(b0, acc.at[pl.ds(sid * rps + r * CHUNK, CHUNK)])
    rem = rps % CHUNK
    if rem:
        pltpu.sync_copy(
            b0.at[pl.ds(0, rem)],
            acc.at[pl.ds(sid * rps + (rps // CHUNK) * CHUNK, rem)])
    plsc.subcore_barrier()

    def gather(j, buf, sem):
        return pltpu.async_copy(featsrc_hbm.at[src_v.at[j]], buf, sem)

    def gwait(j, buf, sem):
        pltpu.make_async_copy(featsrc_hbm.at[src_v.at[j]], buf, sem).wait()

    def scatter(j, buf, sem):
        return pltpu.async_copy(buf, acc.at[dst_v.at[j]], sem, add=True)

    def swait(j, buf, sem):
        pltpu.make_async_copy(buf, acc.at[dst_v.at[j]], sem).wait()

    def window(w, carry):
        pltpu.sync_copy(edges_hbm.at[0, wid, w], src_v)
        pltpu.sync_copy(edges_hbm.at[1, wid, w], dst_v)
        for k in range(NBUF):
            gather(k, bufs[k], gsem[k])

        jt = wch - 1

        def rnd(r, carry2):
            for k in range(NBUF):
                j = r * NBUF + k
                gwait(j, bufs[k], gsem[k])
                scatter(j, bufs[k], ssem[k])
            for k in range(NBUF):
                j = r * NBUF + k
                swait(j, bufs[k], ssem[k])
                if k == 0:
                    @pl.when(r == nround - 1)
                    def _():
                        gather(jt, b0, g0)

                @pl.when(r < nround - 1)
                def _():
                    gather(j + NBUF, bufs[k], gsem[k])

            return carry2

        lax.fori_loop(0, nround, rnd, 0)
        gwait(jt, b0, g0)
        scatter(jt, b0, s0)
        swait(jt, b0, s0)
        return carry

    lax.fori_loop(0, nwin, window, 0)
    plsc.subcore_barrier()
    pltpu.sync_copy(acc.at[pl.ds(sid * rps, rps)],
                    out_hbm.at[cid, pl.ds(sid * rps, rps)])


def _scale_body(cnt_ref, feat_ref, wt_ref, out_ref):
    src_cnt = cnt_ref[0, 0, :] + cnt_ref[1, 0, :]
    ns = lax.rsqrt(jnp.maximum(src_cnt, 1.0))
    y = jnp.dot(feat_ref[...], wt_ref[...],
                preferred_element_type=jnp.float32)
    out_ref[...] = y * ns[:, None]


def _final_body(agg_ref, cnt_ref, feat_ref, b_ref, out_ref):
    a = agg_ref[0] + agg_ref[1]
    dst_cnt = cnt_ref[0, 1, :] + cnt_ref[1, 1, :]
    nd = lax.rsqrt(jnp.maximum(dst_cnt, 1.0))[:, None]
    out_ref[...] = (a + b_ref[...]) * nd + feat_ref[...]


def kernel(feat, edge_index, W, b):
    n, d = feat.shape
    e = edge_index.shape[1]

    n_pad = -(-(n + 1) // ROW_BLK) * ROW_BLK
    assert e % (NW * CHUNK) == 0, "edge count must tile over 32x80"
    nch = e // (NW * CHUNK)
    n_acc = -(-(n + 1) // 128) * 128

    nwin = 5
    assert nch % nwin == 0 and (nch // nwin) % NBUF == 1
    wch = nch // nwin
    edges = edge_index.astype(jnp.int32).reshape(2, NW, nwin, wch, CHUNK)

    mesh = plsc.VectorSubcoreMesh(core_axis_name="c", subcore_axis_name="s")

    count_k = pl.kernel(
        functools.partial(_count_body, nwin=nwin, wch=wch, n_pad=n_pad),
        out_type=jax.ShapeDtypeStruct((NC, 2, n_pad), jnp.float32),
        mesh=mesh,
        scratch_types=[
            pltpu.VMEM((nwin, wch, CHUNK), jnp.int32),
            pltpu.VMEM((nwin, wch, CHUNK), jnp.int32),
            pltpu.VMEM((2 * CHUNK,), jnp.float32),
            pltpu.VMEM_SHARED((n_pad,), jnp.float32),
            pltpu.VMEM_SHARED((n_pad,), jnp.float32),
            pltpu.SemaphoreType.DMA,
            pltpu.SemaphoreType.DMA,
        ],
    )
    cnt = count_k(edges)

    grid = n_pad // ROW_BLK
    feat_src = pl.pallas_call(
        _scale_body,
        grid=(grid,),
        in_specs=[
            pl.BlockSpec((NC, 2, ROW_BLK), lambda i: (0, 0, i)),
            pl.BlockSpec((ROW_BLK, d), lambda i: (i, 0)),
            pl.BlockSpec((d, d), lambda i: (0, 0)),
        ],
        out_specs=pl.BlockSpec((ROW_BLK, d), lambda i: (i, 0)),
        out_shape=jax.ShapeDtypeStruct((n_pad, d), jnp.float32),
    )(cnt, feat, W.T)

    agg_k = pl.kernel(
        functools.partial(_agg_body, nwin=nwin, wch=wch, n_acc=n_acc, d=d),
        out_type=jax.ShapeDtypeStruct((NC, n_acc, d), jnp.float32),
        mesh=mesh,
        scratch_types=[
            pltpu.VMEM((wch, CHUNK), jnp.int32),
            pltpu.VMEM((wch, CHUNK), jnp.int32),
            pltpu.VMEM((CHUNK, d), jnp.float32),
            pltpu.VMEM((CHUNK, d), jnp.float32),
            pltpu.VMEM((CHUNK, d), jnp.float32),
            pltpu.VMEM((CHUNK, d), jnp.float32),
            pltpu.VMEM_SHARED((n_acc, d), jnp.float32),
            pltpu.SemaphoreType.DMA,
            pltpu.SemaphoreType.DMA,
            pltpu.SemaphoreType.DMA,
            pltpu.SemaphoreType.DMA,
            pltpu.SemaphoreType.DMA,
            pltpu.SemaphoreType.DMA,
            pltpu.SemaphoreType.DMA,
            pltpu.SemaphoreType.DMA,
        ],
    )
    agg = agg_k(feat_src, edges)

    return pl.pallas_call(
        _final_body,
        grid=(grid,),
        in_specs=[
            pl.BlockSpec((NC, ROW_BLK, d), lambda i: (0, i, 0)),
            pl.BlockSpec((NC, 2, ROW_BLK), lambda i: (0, 0, i)),
            pl.BlockSpec((ROW_BLK, d), lambda i: (i, 0)),
            pl.BlockSpec((1, d), lambda i: (0, 0)),
        ],
        out_specs=pl.BlockSpec((ROW_BLK, d), lambda i: (i, 0)),
        out_shape=jax.ShapeDtypeStruct((n, d), jnp.float32),
    )(agg, cnt, feat, b.reshape(1, d))
